# baseline (speedup 1.0000x reference)
import jax
import jax.numpy as jnp
from jax import lax
from jax.experimental import pallas as pl
from jax.experimental.pallas import tpu as pltpu
from jax.experimental.pallas import tpu_sc as plsc

B = 4096
NCAT = 26
VCAT = 100000
VSEQ = 1000000
L = 50
D = 32
NCONT = 13
HID = 128

NC = 2
NS = 16
NW = NC * NS
BPW = B // NW
CHUNK = 8
NCHUNK = BPW // CHUNK
CATN = CHUNK * NCAT
SEQN = CHUNK * L
W = 128

SEQ_FT = VSEQ // W
CAT_FT = (VCAT // W)
NUNITS = SEQ_FT + NCAT * CAT_FT
SEQ_EDGE_V = SEQ_FT * W
CAT_EDGE_V = CAT_FT * W
QF = VCAT // 4


def _tp_tile(tile_v, out_v, nrows):
  iota = lax.iota(jnp.int32, 16)

  def row(qq, carry):
    for m in range(8):
      rows = iota + 16 * (m & 1)
      cols = jnp.full((16,), 0, jnp.int32) + (4 * qq + (m >> 1))
      out_v[qq, pl.ds(16 * m, 16)] = plsc.load_gather(tile_v, [rows, cols])
    return carry

  lax.fori_loop(0, nrows, row, 0)


def _relayout_body(seqT_hbm, catT_hbm, seq128_out, cat128_out,
                   tile_v, out_v, eseq_v, ecat_v, sem):
  wid = lax.axis_index("s") * NC + lax.axis_index("c")

  def unit(i, carry):
    u = wid + i * NW
    is_seq = u < SEQ_FT
    u2 = jnp.where(is_seq, 0, u - SEQ_FT)
    f = u2 // CAT_FT
    t = jnp.where(is_seq, u, u2 - f * CAT_FT)

    @pl.when(is_seq)
    def _():
      pltpu.sync_copy(seqT_hbm.at[:, pl.ds(t * W, W)], tile_v)

    @pl.when(jnp.logical_not(is_seq))
    def _():
      pltpu.sync_copy(catT_hbm.at[f, :, pl.ds(t * W, W)], tile_v)

    _tp_tile(tile_v, out_v, 32)

    @pl.when(is_seq)
    def _():
      pltpu.sync_copy(out_v, seq128_out.at[pl.ds(t * 32, 32)])

    @pl.when(jnp.logical_not(is_seq))
    def _():
      pltpu.sync_copy(out_v, cat128_out.at[pl.ds(f * QF + t * 32, 32)])

    return carry

  cnt = (NUNITS - wid + NW - 1) // NW
  lax.fori_loop(0, cnt, unit, 0)

  @pl.when(wid == 0)
  def _():
    pltpu.sync_copy(seqT_hbm.at[:, pl.ds(SEQ_EDGE_V, VSEQ - SEQ_EDGE_V)],
                    eseq_v)
    _tp_tile(eseq_v, out_v, (VSEQ - SEQ_EDGE_V) // 4)
    pltpu.sync_copy(out_v.at[pl.ds(0, (VSEQ - SEQ_EDGE_V) // 4)],
                    seq128_out.at[pl.ds(SEQ_EDGE_V // 4,
                                        (VSEQ - SEQ_EDGE_V) // 4)])

  @pl.when((wid >= 1) & (wid <= NCAT))
  def _():
    f = wid - 1
    ne = VCAT - CAT_EDGE_V
    pltpu.sync_copy(catT_hbm.at[f, :, pl.ds(CAT_EDGE_V, ne)], ecat_v)
    _tp_tile(ecat_v, out_v, ne // 4)
    pltpu.sync_copy(out_v.at[pl.ds(0, ne // 4)],
                    cat128_out.at[pl.ds(f * QF + CAT_EDGE_V // 4, ne // 4)])


def _gather_body(xcat_hbm, hist_hbm, cat128_hbm, seq128_hbm,
                 catrows_out, pooled_out,
                 offs_v, xcat_v, cidx_v, hist_v, sidx_v,
                 catw_v, seqw_v, catrows_v, pooled_v, sem):
  wid = lax.axis_index("s") * NC + lax.axis_index("c")
  base = wid * BPW

  for j in range(CATN // 16):
    pos = lax.iota(jnp.int32, 16) + 16 * j
    offs_v[pl.ds(16 * j, 16)] = lax.rem(pos, NCAT) * VCAT

  def chunk_body(c, carry):
    b0 = base + c * CHUNK
    d1 = pltpu.make_async_copy(
        xcat_hbm.at[pl.ds(b0 * NCAT, CATN)], xcat_v, sem)
    d1.start()
    d2 = pltpu.make_async_copy(
        hist_hbm.at[pl.ds(b0 * L, SEQN)], hist_v, sem)
    d2.start()
    d1.wait()
    d2.wait()

    for j in range(CATN // 16):
      s = pl.ds(16 * j, 16)
      cidx_v[s] = lax.shift_right_logical(xcat_v[s] + offs_v[s], 2)
    for j in range(SEQN // 16):
      s = pl.ds(16 * j, 16)
      sidx_v[s] = lax.shift_right_logical(hist_v[s], 2)

    descs = [
        pltpu.make_async_copy(
            cat128_hbm.at[cidx_v.at[pl.ds(0, 128)]],
            catw_v.at[pl.ds(0, 128)], sem),
        pltpu.make_async_copy(
            cat128_hbm.at[cidx_v.at[pl.ds(128, 80)]],
            catw_v.at[pl.ds(128, 80)], sem),
    ]
    for g in range(3):
      descs.append(pltpu.make_async_copy(
          seq128_hbm.at[sidx_v.at[pl.ds(128 * g, 128)]],
          seqw_v.at[pl.ds(128 * g, 128)], sem))
    descs.append(pltpu.make_async_copy(
        seq128_hbm.at[sidx_v.at[pl.ds(384, 16)]],
        seqw_v.at[pl.ds(384, 16)], sem))
    for d in descs:
      d.start()
    for d in descs:
      d.wait()

    def extract_cat(g, carry2):
      i0 = g * 16
      offv = lax.shift_left(xcat_v[pl.ds(i0, 16)] & 3, 5)
      for u in range(16):
        off = offv[u]
        catrows_v[i0 + u, pl.ds(0, 16)] = catw_v[i0 + u, pl.ds(off, 16)]
        catrows_v[i0 + u, pl.ds(16, 16)] = (
            catw_v[i0 + u, pl.ds(off + 16, 16)])
      return carry2

    lax.fori_loop(0, CATN // 16, extract_cat, 0)

    def pool_b(b, carry2):
      r0 = b * L
      z = jnp.zeros((16,), jnp.float32)
      a0, a1 = z, z
      for g, n in ((0, 16), (1, 16), (2, 16), (3, 2)):
        offv = lax.shift_left(hist_v[pl.ds(r0 + 16 * g, 16)] & 3, 5)
        for u in range(n):
          off = offv[u]
          r = r0 + 16 * g + u
          a0 = a0 + seqw_v[r, pl.ds(off, 16)]
          a1 = a1 + seqw_v[r, pl.ds(off + 16, 16)]
      pooled_v[b, pl.ds(0, 16)] = a0 * (1.0 / L)
      pooled_v[b, pl.ds(16, 16)] = a1 * (1.0 / L)
      return carry2

    lax.fori_loop(0, CHUNK, pool_b, 0)

    pltpu.sync_copy(catrows_v, catrows_out.at[pl.ds(b0 * NCAT, CATN)])
    pltpu.sync_copy(pooled_v, pooled_out.at[pl.ds(b0, CHUNK)])
    return carry

  lax.fori_loop(0, NCHUNK, chunk_body, 0)


def _mlp_body(x1_ref, xc_ref, xp_ref, w1a_ref, w1b_ref, w1c_ref,
              b1_ref, w2_ref, b2_ref, out_ref):
  h = jnp.dot(x1_ref[...], w1a_ref[...], preferred_element_type=jnp.float32)
  h = h + jnp.dot(xc_ref[...], w1b_ref[...],
                  preferred_element_type=jnp.float32)
  h = h + jnp.dot(xp_ref[...], w1c_ref[...],
                  preferred_element_type=jnp.float32)
  h = jax.nn.relu(h + b1_ref[...])
  out = jnp.dot(h, w2_ref[...], preferred_element_type=jnp.float32)
  out_ref[...] = out + b2_ref[0, 0]


def kernel(x_cat, x_cont, hist_seq, cat_tables, seq_table, W1, b1, W2, b2):
  xcat_flat = x_cat.reshape(-1)
  hist_flat = hist_seq.reshape(-1)
  seqT = jnp.transpose(seq_table)
  catT = jnp.transpose(cat_tables, (0, 2, 1))

  mesh = plsc.VectorSubcoreMesh(core_axis_name="c", subcore_axis_name="s")
  relayout = pl.kernel(
      _relayout_body,
      out_type=(
          jax.ShapeDtypeStruct((VSEQ // 4, W), jnp.float32),
          jax.ShapeDtypeStruct((NCAT * VCAT // 4, W), jnp.float32),
      ),
      mesh=mesh,
      compiler_params=pltpu.CompilerParams(needs_layout_passes=False),
      scratch_types=[
          pltpu.VMEM((D, W), jnp.float32),
          pltpu.VMEM((D, W), jnp.float32),
          pltpu.VMEM((D, VSEQ - SEQ_EDGE_V), jnp.float32),
          pltpu.VMEM((D, VCAT - CAT_EDGE_V), jnp.float32),
          pltpu.SemaphoreType.DMA,
      ],
  )
  seq128, cat128 = relayout(seqT, catT)

  gather = pl.kernel(
      _gather_body,
      out_type=(
          jax.ShapeDtypeStruct((B * NCAT, D), jnp.float32),
          jax.ShapeDtypeStruct((B, D), jnp.float32),
      ),
      mesh=mesh,
      compiler_params=pltpu.CompilerParams(use_tc_tiling_on_sc=False),
      scratch_types=[
          pltpu.VMEM((CATN,), jnp.int32),
          pltpu.VMEM((CATN,), jnp.int32),
          pltpu.VMEM((CATN,), jnp.int32),
          pltpu.VMEM((SEQN,), jnp.int32),
          pltpu.VMEM((SEQN,), jnp.int32),
          pltpu.VMEM((CATN, W), jnp.float32),
          pltpu.VMEM((SEQN, W), jnp.float32),
          pltpu.VMEM((CATN, D), jnp.float32),
          pltpu.VMEM((CHUNK, D), jnp.float32),
          pltpu.SemaphoreType.DMA,
      ],
  )
  catrows, pooled = gather(xcat_flat, hist_flat, cat128, seq128)
  cat_flat = catrows.reshape(B, NCAT * D)

  w1a = W1[: NCAT * D]
  w1b = W1[NCAT * D: NCAT * D + NCONT]
  w1c = W1[NCAT * D + NCONT:]
  b1r = b1.reshape(1, HID)
  b2r = b2.reshape(1, 1)

  bm = 512
  grid = (B // bm,)
  logits = pl.pallas_call(
      _mlp_body,
      grid=grid,
      in_specs=[
          pl.BlockSpec((bm, NCAT * D), lambda i: (i, 0)),
          pl.BlockSpec((bm, NCONT), lambda i: (i, 0)),
          pl.BlockSpec((bm, D), lambda i: (i, 0)),
          pl.BlockSpec((NCAT * D, HID), lambda i: (0, 0)),
          pl.BlockSpec((NCONT, HID), lambda i: (0, 0)),
          pl.BlockSpec((D, HID), lambda i: (0, 0)),
          pl.BlockSpec((1, HID), lambda i: (0, 0)),
          pl.BlockSpec((HID, 1), lambda i: (0, 0)),
          pl.BlockSpec((1, 1), lambda i: (0, 0)),
      ],
      out_specs=pl.BlockSpec((bm, 1), lambda i: (i, 0)),
      out_shape=jax.ShapeDtypeStruct((B, 1), jnp.float32),
  )(cat_flat, x_cont, pooled, w1a, w1b, w1c, b1r, W2, b2r)
  return logits.reshape(B)


# speedup vs baseline: 1.3243x; 1.3243x over previous
import jax
import jax.numpy as jnp
from jax import lax
from jax.experimental import pallas as pl
from jax.experimental.pallas import tpu as pltpu
from jax.experimental.pallas import tpu_sc as plsc

B = 4096
NCAT = 26
VCAT = 100000
VSEQ = 1000000
L = 50
D = 32
NCONT = 13
HID = 128

NC = 2
NS = 16
NW = NC * NS
BPW = B // NW
CHUNK = 8
NCHUNK = BPW // CHUNK
CATN = CHUNK * NCAT
SEQN = CHUNK * L
W = 128

SEQ_FT = VSEQ // W
CAT_FT = (VCAT // W)
UV = 512
SEQ_U = SEQ_FT // 4
CAT_U = CAT_FT // 4
NTOT = SEQ_U + NCAT * CAT_U
SEQ_EDGE_V = SEQ_FT * W
CAT_TAIL_V = CAT_U * UV
QF = VCAT // 4


def _tp_tile(tile_v, out_v, nrows):
  iota = lax.iota(jnp.int32, 16)

  def row(qq, carry):
    for m in range(8):
      rows = iota + 16 * (m & 1)
      cols = jnp.full((16,), 0, jnp.int32) + (4 * qq + (m >> 1))
      out_v[qq, pl.ds(16 * m, 16)] = plsc.load_gather(tile_v, [rows, cols])
    return carry

  lax.fori_loop(0, nrows, row, 0)


def _tp_unit(tile_v, out_v):
  iota = lax.iota(jnp.int32, 16)

  def row4(j, carry):
    for r in range(4):
      qq = j * 4 + r
      for m in range(8):
        rows = iota + 16 * (m & 1)
        cols = jnp.full((16,), 0, jnp.int32) + (4 * qq + (m >> 1))
        out_v[qq, pl.ds(16 * m, 16)] = plsc.load_gather(
            tile_v, [rows, cols])
    return carry

  lax.fori_loop(0, 32, row4, 0)


def _relayout_body(seqT_hbm, catT_hbm, seq128_out, cat128_out,
                   in0_v, in1_v, out0_v, out1_v,
                   etile_v, eseq_v, ecat_v,
                   isem0, isem1, osem0, osem1):
  wid = lax.axis_index("s") * NC + lax.axis_index("c")
  cnt = (NTOT - wid + NW - 1) // NW
  inb = (in0_v, in1_v)
  outb = (out0_v, out1_v)
  isem = (isem0, isem1)
  osem = (osem0, osem1)

  def decode(k):
    u = wid + k * NW
    is_seq = u < SEQ_U
    u2 = jnp.where(is_seq, 0, u - SEQ_U)
    f = u2 // CAT_U
    t = jnp.where(is_seq, u, u2 - f * CAT_U)
    q0 = jnp.where(is_seq, t * (UV // 4), f * QF + t * (UV // 4))
    return is_seq, f, t, q0

  def issue_in(k, p):
    is_seq, f, t, _ = decode(k)

    @pl.when(is_seq)
    def _():
      pltpu.make_async_copy(
          seqT_hbm.at[:, pl.ds(t * UV, UV)], inb[p], isem[p]).start()

    @pl.when(jnp.logical_not(is_seq))
    def _():
      pltpu.make_async_copy(
          catT_hbm.at[f, :, pl.ds(t * UV, UV)], inb[p], isem[p]).start()

  def issue_out(k, p):
    is_seq, _, _, q0 = decode(k)

    @pl.when(is_seq)
    def _():
      pltpu.make_async_copy(
          outb[p], seq128_out.at[pl.ds(q0, UV // 4)], osem[p]).start()

    @pl.when(jnp.logical_not(is_seq))
    def _():
      pltpu.make_async_copy(
          outb[p], cat128_out.at[pl.ds(q0, UV // 4)], osem[p]).start()

  def wait_in(p):
    pltpu.make_async_copy(seqT_hbm.at[:, pl.ds(0, UV)], inb[p],
                          isem[p]).wait()

  def wait_out(p):
    pltpu.make_async_copy(outb[p], seq128_out.at[pl.ds(0, UV // 4)],
                          osem[p]).wait()

  issue_in(0, 0)

  def pair(j, carry):
    for p in (0, 1):
      k = j * 2 + p

      @pl.when(k < cnt)
      def _():
        @pl.when(k + 1 < cnt)
        def _():
          issue_in(k + 1, 1 - p)

        wait_in(p)

        @pl.when(k >= 2)
        def _():
          wait_out(p)

        _tp_unit(inb[p], outb[p])
        issue_out(k, p)

    return carry

  lax.fori_loop(0, (cnt + 1) // 2, pair, 0)
  wait_out(0)
  wait_out(1)

  @pl.when(wid == 0)
  def _():
    ne = VSEQ - SEQ_EDGE_V
    pltpu.sync_copy(seqT_hbm.at[:, pl.ds(SEQ_EDGE_V, ne)], eseq_v)
    _tp_tile(eseq_v, out0_v, ne // 4)
    pltpu.sync_copy(out0_v.at[pl.ds(0, ne // 4)],
                    seq128_out.at[pl.ds(SEQ_EDGE_V // 4, ne // 4)])

  @pl.when((wid >= 1) & (wid <= NCAT))
  def _():
    f = wid - 1
    pltpu.sync_copy(catT_hbm.at[f, :, pl.ds(CAT_TAIL_V, W)], etile_v)
    _tp_tile(etile_v, out1_v, 32)
    pltpu.sync_copy(out1_v.at[pl.ds(0, 32)],
                    cat128_out.at[pl.ds(f * QF + CAT_TAIL_V // 4, 32)])
    ne = VCAT - CAT_TAIL_V - W
    pltpu.sync_copy(catT_hbm.at[f, :, pl.ds(CAT_TAIL_V + W, ne)], ecat_v)
    _tp_tile(ecat_v, out1_v, ne // 4)
    pltpu.sync_copy(
        out1_v.at[pl.ds(0, ne // 4)],
        cat128_out.at[pl.ds(f * QF + (CAT_TAIL_V + W) // 4, ne // 4)])


def _gather_body(xcat_hbm, hist_hbm, cat128_hbm, seq128_hbm,
                 catrows_out, pooled_out,
                 offs_v, xcat_v, cidx_v, hist_v, sidx_v,
                 catw_v, seqw_v, catrows_v, pooled_v, sem):
  wid = lax.axis_index("s") * NC + lax.axis_index("c")
  base = wid * BPW

  for j in range(CATN // 16):
    pos = lax.iota(jnp.int32, 16) + 16 * j
    offs_v[pl.ds(16 * j, 16)] = lax.rem(pos, NCAT) * VCAT

  def chunk_body(c, carry):
    b0 = base + c * CHUNK
    d1 = pltpu.make_async_copy(
        xcat_hbm.at[pl.ds(b0 * NCAT, CATN)], xcat_v, sem)
    d1.start()
    d2 = pltpu.make_async_copy(
        hist_hbm.at[pl.ds(b0 * L, SEQN)], hist_v, sem)
    d2.start()
    d1.wait()
    d2.wait()

    for j in range(CATN // 16):
      s = pl.ds(16 * j, 16)
      cidx_v[s] = lax.shift_right_logical(xcat_v[s] + offs_v[s], 2)
    for j in range(SEQN // 16):
      s = pl.ds(16 * j, 16)
      sidx_v[s] = lax.shift_right_logical(hist_v[s], 2)

    descs = [
        pltpu.make_async_copy(
            cat128_hbm.at[cidx_v.at[pl.ds(0, 128)]],
            catw_v.at[pl.ds(0, 128)], sem),
        pltpu.make_async_copy(
            cat128_hbm.at[cidx_v.at[pl.ds(128, 80)]],
            catw_v.at[pl.ds(128, 80)], sem),
    ]
    for g in range(3):
      descs.append(pltpu.make_async_copy(
          seq128_hbm.at[sidx_v.at[pl.ds(128 * g, 128)]],
          seqw_v.at[pl.ds(128 * g, 128)], sem))
    descs.append(pltpu.make_async_copy(
        seq128_hbm.at[sidx_v.at[pl.ds(384, 16)]],
        seqw_v.at[pl.ds(384, 16)], sem))
    for d in descs:
      d.start()
    for d in descs:
      d.wait()

    def extract_cat(g, carry2):
      i0 = g * 16
      offv = lax.shift_left(xcat_v[pl.ds(i0, 16)] & 3, 5)
      for u in range(16):
        off = offv[u]
        catrows_v[i0 + u, pl.ds(0, 16)] = catw_v[i0 + u, pl.ds(off, 16)]
        catrows_v[i0 + u, pl.ds(16, 16)] = (
            catw_v[i0 + u, pl.ds(off + 16, 16)])
      return carry2

    lax.fori_loop(0, CATN // 16, extract_cat, 0)

    def pool_b(b, carry2):
      r0 = b * L
      z = jnp.zeros((16,), jnp.float32)
      a0, a1 = z, z
      for g, n in ((0, 16), (1, 16), (2, 16), (3, 2)):
        offv = lax.shift_left(hist_v[pl.ds(r0 + 16 * g, 16)] & 3, 5)
        for u in range(n):
          off = offv[u]
          r = r0 + 16 * g + u
          a0 = a0 + seqw_v[r, pl.ds(off, 16)]
          a1 = a1 + seqw_v[r, pl.ds(off + 16, 16)]
      pooled_v[b, pl.ds(0, 16)] = a0 * (1.0 / L)
      pooled_v[b, pl.ds(16, 16)] = a1 * (1.0 / L)
      return carry2

    lax.fori_loop(0, CHUNK, pool_b, 0)

    pltpu.sync_copy(catrows_v, catrows_out.at[pl.ds(b0 * NCAT, CATN)])
    pltpu.sync_copy(pooled_v, pooled_out.at[pl.ds(b0, CHUNK)])
    return carry

  lax.fori_loop(0, NCHUNK, chunk_body, 0)


def _mlp_body(x1_ref, xc_ref, xp_ref, w1a_ref, w1b_ref, w1c_ref,
              b1_ref, w2_ref, b2_ref, out_ref):
  h = jnp.dot(x1_ref[...], w1a_ref[...], preferred_element_type=jnp.float32)
  h = h + jnp.dot(xc_ref[...], w1b_ref[...],
                  preferred_element_type=jnp.float32)
  h = h + jnp.dot(xp_ref[...], w1c_ref[...],
                  preferred_element_type=jnp.float32)
  h = jax.nn.relu(h + b1_ref[...])
  out = jnp.dot(h, w2_ref[...], preferred_element_type=jnp.float32)
  out_ref[...] = out + b2_ref[0, 0]


def kernel(x_cat, x_cont, hist_seq, cat_tables, seq_table, W1, b1, W2, b2):
  xcat_flat = x_cat.reshape(-1)
  hist_flat = hist_seq.reshape(-1)
  seqT = jnp.transpose(seq_table)
  catT = jnp.transpose(cat_tables, (0, 2, 1))

  mesh = plsc.VectorSubcoreMesh(core_axis_name="c", subcore_axis_name="s")
  relayout = pl.kernel(
      _relayout_body,
      out_type=(
          jax.ShapeDtypeStruct((VSEQ // 4, W), jnp.float32),
          jax.ShapeDtypeStruct((NCAT * VCAT // 4, W), jnp.float32),
      ),
      mesh=mesh,
      compiler_params=pltpu.CompilerParams(needs_layout_passes=False),
      scratch_types=[
          pltpu.VMEM((D, UV), jnp.float32),
          pltpu.VMEM((D, UV), jnp.float32),
          pltpu.VMEM((UV // 4, W), jnp.float32),
          pltpu.VMEM((UV // 4, W), jnp.float32),
          pltpu.VMEM((D, W), jnp.float32),
          pltpu.VMEM((D, VSEQ - SEQ_EDGE_V), jnp.float32),
          pltpu.VMEM((D, VCAT - CAT_TAIL_V - W), jnp.float32),
          pltpu.SemaphoreType.DMA,
          pltpu.SemaphoreType.DMA,
          pltpu.SemaphoreType.DMA,
          pltpu.SemaphoreType.DMA,
      ],
  )
  seq128, cat128 = relayout(seqT, catT)

  gather = pl.kernel(
      _gather_body,
      out_type=(
          jax.ShapeDtypeStruct((B * NCAT, D), jnp.float32),
          jax.ShapeDtypeStruct((B, D), jnp.float32),
      ),
      mesh=mesh,
      compiler_params=pltpu.CompilerParams(use_tc_tiling_on_sc=False),
      scratch_types=[
          pltpu.VMEM((CATN,), jnp.int32),
          pltpu.VMEM((CATN,), jnp.int32),
          pltpu.VMEM((CATN,), jnp.int32),
          pltpu.VMEM((SEQN,), jnp.int32),
          pltpu.VMEM((SEQN,), jnp.int32),
          pltpu.VMEM((CATN, W), jnp.float32),
          pltpu.VMEM((SEQN, W), jnp.float32),
          pltpu.VMEM((CATN, D), jnp.float32),
          pltpu.VMEM((CHUNK, D), jnp.float32),
          pltpu.SemaphoreType.DMA,
      ],
  )
  catrows, pooled = gather(xcat_flat, hist_flat, cat128, seq128)
  cat_flat = catrows.reshape(B, NCAT * D)

  w1a = W1[: NCAT * D]
  w1b = W1[NCAT * D: NCAT * D + NCONT]
  w1c = W1[NCAT * D + NCONT:]
  b1r = b1.reshape(1, HID)
  b2r = b2.reshape(1, 1)

  bm = 512
  grid = (B // bm,)
  logits = pl.pallas_call(
      _mlp_body,
      grid=grid,
      in_specs=[
          pl.BlockSpec((bm, NCAT * D), lambda i: (i, 0)),
          pl.BlockSpec((bm, NCONT), lambda i: (i, 0)),
          pl.BlockSpec((bm, D), lambda i: (i, 0)),
          pl.BlockSpec((NCAT * D, HID), lambda i: (0, 0)),
          pl.BlockSpec((NCONT, HID), lambda i: (0, 0)),
          pl.BlockSpec((D, HID), lambda i: (0, 0)),
          pl.BlockSpec((1, HID), lambda i: (0, 0)),
          pl.BlockSpec((HID, 1), lambda i: (0, 0)),
          pl.BlockSpec((1, 1), lambda i: (0, 0)),
      ],
      out_specs=pl.BlockSpec((bm, 1), lambda i: (i, 0)),
      out_shape=jax.ShapeDtypeStruct((B, 1), jnp.float32),
  )(cat_flat, x_cont, pooled, w1a, w1b, w1c, b1r, W2, b2r)
  return logits.reshape(B)
